# baseline (speedup 1.0000x reference)
import functools

import jax
import jax.numpy as jnp
from jax import lax
from jax.experimental import pallas as pl
from jax.experimental.pallas import tpu as pltpu
from jax.experimental.pallas import tpu_sc as plsc

N_NODES = 10000
N_EDGES = 320000
DIM = 128
BATCH = 4096

NC = 2
NS = 16
NW = NC * NS

K = 128
CH_PER_W = 80
NCHUNK = CH_PER_W * NW
E_PAD = NCHUNK * K

NPAD = 10112
ROWS_PER_TILE = NPAD // NS

NBUF = 2
NHALF = 2
M = CH_PER_W // NHALF

_mesh = plsc.VectorSubcoreMesh(core_axis_name="c", subcore_axis_name="s")


@functools.partial(
    pl.kernel,
    mesh=_mesh,
    out_type=jax.ShapeDtypeStruct((NC, NPAD, DIM), jnp.float32),
    scratch_types=[
        pltpu.VMEM((M, K), jnp.int32),
        pltpu.VMEM((M, K), jnp.int32),
        pltpu.VMEM((NBUF, K, DIM), jnp.float32),
        pltpu.VMEM_SHARED((NPAD, DIM), jnp.float32),
        pltpu.SemaphoreType.DMA((NBUF,)),
        pltpu.SemaphoreType.DMA((NBUF,)),
    ],
)
def _sc_segment_sum(table, srcs, dsts, zeros, out, src_v, dst_v, rows_v, acc,
                    gsem, ssem):
    c = lax.axis_index("c")
    s = lax.axis_index("s")
    wid = c * NS + s

    pltpu.sync_copy(zeros.at[pl.ds(s * ROWS_PER_TILE, ROWS_PER_TILE)],
                    acc.at[pl.ds(s * ROWS_PER_TILE, ROWS_PER_TILE)])
    plsc.subcore_barrier()

    for h in range(NHALF):
        pltpu.sync_copy(srcs.at[pl.ds(wid * CH_PER_W + h * M, M)], src_v)
        pltpu.sync_copy(dsts.at[pl.ds(wid * CH_PER_W + h * M, M)], dst_v)

        def step(j, carry):
            b = lax.rem(j, NBUF)

            @pl.when(j >= 1)
            def _scatter():
                b2 = lax.rem(j - 1, NBUF)
                pltpu.make_async_copy(table.at[src_v.at[j - 1]],
                                      rows_v.at[b2], gsem.at[b2]).wait()
                pltpu.async_copy(rows_v.at[b2], acc.at[dst_v.at[j - 1]],
                                 ssem.at[b2], add=True)

            @pl.when(j < M)
            def _gather():
                @pl.when(j >= NBUF)
                def _free():
                    pltpu.make_async_copy(rows_v.at[b],
                                          acc.at[dst_v.at[j - NBUF]],
                                          ssem.at[b]).wait()

                pltpu.async_copy(table.at[src_v.at[j]], rows_v.at[b],
                                 gsem.at[b])

            return carry

        lax.fori_loop(0, M + 1, step, 0)
        for bb in range(NBUF):
            pltpu.make_async_copy(rows_v.at[bb], acc.at[dst_v.at[0]],
                                  ssem.at[bb]).wait()

    plsc.subcore_barrier()
    pltpu.sync_copy(acc.at[pl.ds(s * ROWS_PER_TILE, ROWS_PER_TILE)],
                    out.at[c, pl.ds(s * ROWS_PER_TILE, ROWS_PER_TILE)])


B_PER_W = BATCH // NW


@functools.partial(
    pl.kernel,
    mesh=_mesh,
    out_type=[
        jax.ShapeDtypeStruct((BATCH, DIM), jnp.float32),
        jax.ShapeDtypeStruct((BATCH, DIM), jnp.float32),
        jax.ShapeDtypeStruct((BATCH, DIM), jnp.float32),
        jax.ShapeDtypeStruct((BATCH, DIM), jnp.float32),
    ],
    scratch_types=[
        pltpu.VMEM((6, B_PER_W), jnp.int32),
        pltpu.VMEM((2, B_PER_W, DIM), jnp.float32),
        pltpu.SemaphoreType.DMA((6,)),
        pltpu.SemaphoreType.DMA((2,)),
        pltpu.SemaphoreType.DMA((2,)),
    ],
)
def _sc_gather(p2, rel_emb, norm_emb, heads_lo, heads_hi, tails_lo, tails_hi,
               rels, times, out_h, out_t, out_r, out_nv, idx_v, buf,
               isem, gsem, wsem):
    c = lax.axis_index("c")
    s = lax.axis_index("s")
    wid = c * NS + s
    base = wid * B_PER_W
    idx_src = (heads_lo, heads_hi, tails_lo, tails_hi, rels, times)
    for k, idx_hbm in enumerate(idx_src):
        pltpu.async_copy(idx_hbm.at[pl.ds(base, B_PER_W)], idx_v.at[k], isem.at[k])
    plans = ((p2, 0, 1, out_h), (p2, 2, 3, out_t),
             (rel_emb, 4, None, out_r), (norm_emb, 5, None, out_nv))
    for k, (tbl, i0, i1, dst) in enumerate(plans):
        b = k % 2
        if k >= 2:
            pltpu.make_async_copy(buf.at[b], dst.at[pl.ds(base, B_PER_W)],
                                  wsem.at[b]).wait()
        pltpu.make_async_copy(idx_src[i0].at[pl.ds(base, B_PER_W)],
                              idx_v.at[i0], isem.at[i0]).wait()
        pltpu.async_copy(tbl.at[idx_v.at[i0]], buf.at[b], gsem.at[b]).wait()
        if i1 is not None:
            pltpu.make_async_copy(idx_src[i1].at[pl.ds(base, B_PER_W)],
                                  idx_v.at[i1], isem.at[i1]).wait()
            pltpu.async_copy(tbl.at[idx_v.at[i1]], buf.at[b], gsem.at[b],
                             add=True).wait()
        pltpu.async_copy(buf.at[b], dst.at[pl.ds(base, B_PER_W)], wsem.at[b])
    for b in range(2):
        pltpu.make_async_copy(buf.at[b], out_r.at[pl.ds(base, B_PER_W)],
                              wsem.at[b]).wait()


ACT_BLK = 2000


def _act_body(p_ref, w_ref, b_ref, o_ref):
    agg = p_ref[0] + p_ref[1]
    y = lax.dot_general(agg, w_ref[...], (((1,), (1,)), ((), ())),
                        preferred_element_type=jnp.float32)
    o_ref[...] = jnp.tanh(y + b_ref[...])


_tc_act = pl.pallas_call(
    _act_body,
    grid=(N_NODES // ACT_BLK,),
    in_specs=[
        pl.BlockSpec((NC, ACT_BLK, DIM), lambda i: (0, i, 0)),
        pl.BlockSpec((DIM, DIM), lambda i: (0, 0)),
        pl.BlockSpec((1, DIM), lambda i: (0, 0)),
    ],
    out_specs=pl.BlockSpec((ACT_BLK, DIM), lambda i: (i, 0)),
    out_shape=jax.ShapeDtypeStruct((N_NODES, DIM), jnp.float32),
)

SCORE_BLK = 4096


def _l2n(e, eps=1e-12):
    n = jnp.sqrt(jnp.sum(e * e, axis=-1, keepdims=True))
    return e / jnp.maximum(n, eps)


def _score_body(h_ref, t_ref, r_ref, nv_ref, w_ref, b_ref, o_ref):
    def act(p_ref):
        y = lax.dot_general(p_ref[...], w_ref[...], (((1,), (1,)), ((), ())),
                            preferred_element_type=jnp.float32)
        return jnp.tanh(y + b_ref[...])

    nvn = _l2n(nv_ref[...])

    def proj(e):
        return e - jnp.sum(nvn * e, axis=-1, keepdims=True) * nvn

    h = _l2n(proj(act(h_ref)))
    r = _l2n(proj(r_ref[...]))
    t = _l2n(proj(act(t_ref)))
    d = h + r - t
    o_ref[...] = jnp.sqrt(jnp.sum(d * d, axis=-1, keepdims=True))


_tc_score = pl.pallas_call(
    _score_body,
    grid=(BATCH // SCORE_BLK,),
    in_specs=[
        pl.BlockSpec((SCORE_BLK, DIM), lambda i: (i, 0)),
        pl.BlockSpec((SCORE_BLK, DIM), lambda i: (i, 0)),
        pl.BlockSpec((SCORE_BLK, DIM), lambda i: (i, 0)),
        pl.BlockSpec((SCORE_BLK, DIM), lambda i: (i, 0)),
        pl.BlockSpec((DIM, DIM), lambda i: (0, 0)),
        pl.BlockSpec((1, DIM), lambda i: (0, 0)),
    ],
    out_specs=pl.BlockSpec((SCORE_BLK, 1), lambda i: (i, 0)),
    out_shape=jax.ShapeDtypeStruct((BATCH, 1), jnp.float32),
)


def kernel(x, edge_index, head_batched, rel_batched, tail_batched, time_batched,
           gcn_W, gcn_b, rel_emb, norm_emb):
    src = edge_index[0]
    dst = edge_index[1]
    pad = E_PAD - N_EDGES
    pad_i = jnp.arange(pad, dtype=jnp.int32)
    src_pad = pad_i % N_NODES
    dst_pad = N_NODES + pad_i % (NPAD - N_NODES)
    srcs = jnp.concatenate([src, src_pad]).reshape(NCHUNK, K)
    dsts = jnp.concatenate([dst, dst_pad]).reshape(NCHUNK, K)
    zeros = jnp.zeros((NPAD, DIM), jnp.float32)
    b2 = gcn_b.reshape(1, DIM)

    p1 = _sc_segment_sum(x, srcs, dsts, zeros)
    h1 = _tc_act(p1, gcn_W, b2)
    p2 = _sc_segment_sum(h1, srcs, dsts, zeros)

    h_agg, t_agg, r, nv = _sc_gather(
        p2.reshape(NC * NPAD, DIM), rel_emb, norm_emb,
        head_batched, head_batched + NPAD, tail_batched, tail_batched + NPAD,
        rel_batched, time_batched)
    return _tc_score(h_agg, t_agg, r, nv, gcn_W, b2).reshape(-1)


# speedup vs baseline: 1.1461x; 1.1461x over previous
import functools

import jax
import jax.numpy as jnp
from jax import lax
from jax.experimental import pallas as pl
from jax.experimental.pallas import tpu as pltpu
from jax.experimental.pallas import tpu_sc as plsc

N_NODES = 10000
N_EDGES = 320000
DIM = 128
BATCH = 4096

NC = 2
NS = 16
NW = NC * NS

K = 128
CH_PER_W = 80
NCHUNK = CH_PER_W * NW
E_PAD = NCHUNK * K

NPAD = 10112
ROWS_PER_TILE = NPAD // NS

NBUF = 2
NHALF = 2
M = CH_PER_W // NHALF

_mesh = plsc.VectorSubcoreMesh(core_axis_name="c", subcore_axis_name="s")


@functools.partial(
    pl.kernel,
    mesh=_mesh,
    out_type=jax.ShapeDtypeStruct((NC, NPAD, DIM), jnp.float32),
    scratch_types=[
        pltpu.VMEM((M, K), jnp.int32),
        pltpu.VMEM((M, K), jnp.int32),
        pltpu.VMEM((NBUF, K, DIM), jnp.float32),
        pltpu.VMEM_SHARED((NPAD, DIM), jnp.float32),
        pltpu.SemaphoreType.DMA((NBUF,)),
        pltpu.SemaphoreType.DMA((NBUF,)),
    ],
)
def _sc_segment_sum(table, srcs, dsts, zeros, out, src_v, dst_v, rows_v, acc,
                    gsem, ssem):
    c = lax.axis_index("c")
    s = lax.axis_index("s")
    wid = c * NS + s

    pltpu.sync_copy(zeros.at[pl.ds(s * ROWS_PER_TILE, ROWS_PER_TILE)],
                    acc.at[pl.ds(s * ROWS_PER_TILE, ROWS_PER_TILE)])
    plsc.subcore_barrier()

    for h in range(NHALF):
        pltpu.sync_copy(srcs.at[pl.ds(wid * CH_PER_W + h * M, M)], src_v)
        pltpu.sync_copy(dsts.at[pl.ds(wid * CH_PER_W + h * M, M)], dst_v)

        def step(j, carry):
            b = lax.rem(j, NBUF)

            @pl.when(j < M)
            def _gather():
                @pl.when(j >= NBUF)
                def _free():
                    pltpu.make_async_copy(rows_v.at[b],
                                          acc.at[dst_v.at[j - NBUF]],
                                          ssem.at[b]).wait()

                pltpu.async_copy(table.at[src_v.at[j]], rows_v.at[b],
                                 gsem.at[b])

            @pl.when(j >= 1)
            def _scatter():
                b2 = lax.rem(j - 1, NBUF)
                pltpu.make_async_copy(table.at[src_v.at[j - 1]],
                                      rows_v.at[b2], gsem.at[b2]).wait()
                pltpu.async_copy(rows_v.at[b2], acc.at[dst_v.at[j - 1]],
                                 ssem.at[b2], add=True)

            return carry

        lax.fori_loop(0, M + 1, step, 0)
        for bb in range(NBUF):
            pltpu.make_async_copy(rows_v.at[bb], acc.at[dst_v.at[0]],
                                  ssem.at[bb]).wait()

    plsc.subcore_barrier()
    pltpu.sync_copy(acc.at[pl.ds(s * ROWS_PER_TILE, ROWS_PER_TILE)],
                    out.at[c, pl.ds(s * ROWS_PER_TILE, ROWS_PER_TILE)])


B_PER_W = BATCH // NW


@functools.partial(
    pl.kernel,
    mesh=_mesh,
    out_type=[
        jax.ShapeDtypeStruct((BATCH, DIM), jnp.float32),
        jax.ShapeDtypeStruct((BATCH, DIM), jnp.float32),
        jax.ShapeDtypeStruct((BATCH, DIM), jnp.float32),
        jax.ShapeDtypeStruct((BATCH, DIM), jnp.float32),
    ],
    scratch_types=[
        pltpu.VMEM((6, B_PER_W), jnp.int32),
        pltpu.VMEM((2, B_PER_W, DIM), jnp.float32),
        pltpu.SemaphoreType.DMA((6,)),
        pltpu.SemaphoreType.DMA((2,)),
        pltpu.SemaphoreType.DMA((2,)),
    ],
)
def _sc_gather(p2, rel_emb, norm_emb, heads_lo, heads_hi, tails_lo, tails_hi,
               rels, times, out_h, out_t, out_r, out_nv, idx_v, buf,
               isem, gsem, wsem):
    c = lax.axis_index("c")
    s = lax.axis_index("s")
    wid = c * NS + s
    base = wid * B_PER_W
    idx_src = (heads_lo, heads_hi, tails_lo, tails_hi, rels, times)
    for k, idx_hbm in enumerate(idx_src):
        pltpu.async_copy(idx_hbm.at[pl.ds(base, B_PER_W)], idx_v.at[k], isem.at[k])
    plans = ((p2, 0, 1, out_h), (p2, 2, 3, out_t),
             (rel_emb, 4, None, out_r), (norm_emb, 5, None, out_nv))
    for k, (tbl, i0, i1, dst) in enumerate(plans):
        b = k % 2
        if k >= 2:
            pltpu.make_async_copy(buf.at[b], dst.at[pl.ds(base, B_PER_W)],
                                  wsem.at[b]).wait()
        pltpu.make_async_copy(idx_src[i0].at[pl.ds(base, B_PER_W)],
                              idx_v.at[i0], isem.at[i0]).wait()
        pltpu.async_copy(tbl.at[idx_v.at[i0]], buf.at[b], gsem.at[b]).wait()
        if i1 is not None:
            pltpu.make_async_copy(idx_src[i1].at[pl.ds(base, B_PER_W)],
                                  idx_v.at[i1], isem.at[i1]).wait()
            pltpu.async_copy(tbl.at[idx_v.at[i1]], buf.at[b], gsem.at[b],
                             add=True).wait()
        pltpu.async_copy(buf.at[b], dst.at[pl.ds(base, B_PER_W)], wsem.at[b])
    for b in range(2):
        pltpu.make_async_copy(buf.at[b], out_r.at[pl.ds(base, B_PER_W)],
                              wsem.at[b]).wait()


ACT_BLK = 2000


def _act_body(p_ref, w_ref, b_ref, o_ref):
    agg = p_ref[0] + p_ref[1]
    y = lax.dot_general(agg, w_ref[...], (((1,), (1,)), ((), ())),
                        preferred_element_type=jnp.float32)
    o_ref[...] = jnp.tanh(y + b_ref[...])


_tc_act = pl.pallas_call(
    _act_body,
    grid=(N_NODES // ACT_BLK,),
    in_specs=[
        pl.BlockSpec((NC, ACT_BLK, DIM), lambda i: (0, i, 0)),
        pl.BlockSpec((DIM, DIM), lambda i: (0, 0)),
        pl.BlockSpec((1, DIM), lambda i: (0, 0)),
    ],
    out_specs=pl.BlockSpec((ACT_BLK, DIM), lambda i: (i, 0)),
    out_shape=jax.ShapeDtypeStruct((N_NODES, DIM), jnp.float32),
)

SCORE_BLK = 4096


def _l2n(e, eps=1e-12):
    n = jnp.sqrt(jnp.sum(e * e, axis=-1, keepdims=True))
    return e / jnp.maximum(n, eps)


def _score_body(h_ref, t_ref, r_ref, nv_ref, w_ref, b_ref, o_ref):
    def act(p_ref):
        y = lax.dot_general(p_ref[...], w_ref[...], (((1,), (1,)), ((), ())),
                            preferred_element_type=jnp.float32)
        return jnp.tanh(y + b_ref[...])

    nvn = _l2n(nv_ref[...])

    def proj(e):
        return e - jnp.sum(nvn * e, axis=-1, keepdims=True) * nvn

    h = _l2n(proj(act(h_ref)))
    r = _l2n(proj(r_ref[...]))
    t = _l2n(proj(act(t_ref)))
    d = h + r - t
    o_ref[...] = jnp.sqrt(jnp.sum(d * d, axis=-1, keepdims=True))


_tc_score = pl.pallas_call(
    _score_body,
    grid=(BATCH // SCORE_BLK,),
    in_specs=[
        pl.BlockSpec((SCORE_BLK, DIM), lambda i: (i, 0)),
        pl.BlockSpec((SCORE_BLK, DIM), lambda i: (i, 0)),
        pl.BlockSpec((SCORE_BLK, DIM), lambda i: (i, 0)),
        pl.BlockSpec((SCORE_BLK, DIM), lambda i: (i, 0)),
        pl.BlockSpec((DIM, DIM), lambda i: (0, 0)),
        pl.BlockSpec((1, DIM), lambda i: (0, 0)),
    ],
    out_specs=pl.BlockSpec((SCORE_BLK, 1), lambda i: (i, 0)),
    out_shape=jax.ShapeDtypeStruct((BATCH, 1), jnp.float32),
)


def kernel(x, edge_index, head_batched, rel_batched, tail_batched, time_batched,
           gcn_W, gcn_b, rel_emb, norm_emb):
    src = edge_index[0]
    dst = edge_index[1]
    pad = E_PAD - N_EDGES
    pad_i = jnp.arange(pad, dtype=jnp.int32)
    src_pad = pad_i % N_NODES
    dst_pad = N_NODES + pad_i % (NPAD - N_NODES)
    srcs = jnp.concatenate([src, src_pad]).reshape(NCHUNK, K)
    dsts = jnp.concatenate([dst, dst_pad]).reshape(NCHUNK, K)
    zeros = jnp.zeros((NPAD, DIM), jnp.float32)
    b2 = gcn_b.reshape(1, DIM)

    p1 = _sc_segment_sum(x, srcs, dsts, zeros)
    h1 = _tc_act(p1, gcn_W, b2)
    p2 = _sc_segment_sum(h1, srcs, dsts, zeros)

    h_agg, t_agg, r, nv = _sc_gather(
        p2.reshape(NC * NPAD, DIM), rel_emb, norm_emb,
        head_batched, head_batched + NPAD, tail_batched, tail_batched + NPAD,
        rel_batched, time_batched)
    return _tc_score(h_agg, t_agg, r, nv, gcn_W, b2).reshape(-1)


# speedup vs baseline: 1.1884x; 1.0369x over previous
import functools

import jax
import jax.numpy as jnp
from jax import lax
from jax.experimental import pallas as pl
from jax.experimental.pallas import tpu as pltpu
from jax.experimental.pallas import tpu_sc as plsc

N_NODES = 10000
N_EDGES = 320000
DIM = 128
BATCH = 4096

NC = 2
NS = 16
NW = NC * NS

K = 128
CH_PER_W = 80
NCHUNK = CH_PER_W * NW
E_PAD = NCHUNK * K

NPAD = 10112
ROWS_PER_TILE = NPAD // NS

NBUF = 2
NHALF = 2
M = CH_PER_W // NHALF

_mesh = plsc.VectorSubcoreMesh(core_axis_name="c", subcore_axis_name="s")


@functools.partial(
    pl.kernel,
    mesh=_mesh,
    out_type=jax.ShapeDtypeStruct((NC, NPAD, DIM), jnp.float32),
    scratch_types=[
        pltpu.VMEM((M, K), jnp.int32),
        pltpu.VMEM((M, K), jnp.int32),
        pltpu.VMEM((NBUF, K, DIM), jnp.float32),
        pltpu.VMEM_SHARED((NPAD, DIM), jnp.float32),
        pltpu.SemaphoreType.DMA((NBUF,)),
        pltpu.SemaphoreType.DMA((NBUF,)),
    ],
)
def _sc_segment_sum(table, edges, zeros, out, src_v, dst_v, rows_v, acc,
                    gsem, ssem):
    c = lax.axis_index("c")
    s = lax.axis_index("s")
    wid = c * NS + s

    pltpu.sync_copy(zeros.at[pl.ds(s * ROWS_PER_TILE, ROWS_PER_TILE)],
                    acc.at[pl.ds(s * ROWS_PER_TILE, ROWS_PER_TILE)])
    plsc.subcore_barrier()

    for h in range(NHALF):
        pltpu.sync_copy(edges.at[0, pl.ds(wid * CH_PER_W + h * M, M)], src_v)
        pltpu.sync_copy(edges.at[1, pl.ds(wid * CH_PER_W + h * M, M)], dst_v)

        def step(j, carry):
            b = lax.rem(j, NBUF)

            @pl.when(j < M)
            def _gather():
                @pl.when(j >= NBUF)
                def _free():
                    pltpu.make_async_copy(rows_v.at[b],
                                          acc.at[dst_v.at[j - NBUF]],
                                          ssem.at[b]).wait()

                pltpu.async_copy(table.at[src_v.at[j]], rows_v.at[b],
                                 gsem.at[b])

            @pl.when(j >= 1)
            def _scatter():
                b2 = lax.rem(j - 1, NBUF)
                pltpu.make_async_copy(table.at[src_v.at[j - 1]],
                                      rows_v.at[b2], gsem.at[b2]).wait()
                pltpu.async_copy(rows_v.at[b2], acc.at[dst_v.at[j - 1]],
                                 ssem.at[b2], add=True)

            return carry

        lax.fori_loop(0, M + 1, step, 0)
        for bb in range(NBUF):
            pltpu.make_async_copy(rows_v.at[bb], acc.at[dst_v.at[0]],
                                  ssem.at[bb]).wait()

    plsc.subcore_barrier()
    pltpu.sync_copy(acc.at[pl.ds(s * ROWS_PER_TILE, ROWS_PER_TILE)],
                    out.at[c, pl.ds(s * ROWS_PER_TILE, ROWS_PER_TILE)])


B_PER_W = BATCH // NW


@functools.partial(
    pl.kernel,
    mesh=_mesh,
    out_type=[
        jax.ShapeDtypeStruct((BATCH, DIM), jnp.float32),
        jax.ShapeDtypeStruct((BATCH, DIM), jnp.float32),
        jax.ShapeDtypeStruct((BATCH, DIM), jnp.float32),
        jax.ShapeDtypeStruct((BATCH, DIM), jnp.float32),
    ],
    scratch_types=[
        pltpu.VMEM((6, B_PER_W), jnp.int32),
        pltpu.VMEM((2, B_PER_W, DIM), jnp.float32),
        pltpu.SemaphoreType.DMA((6,)),
        pltpu.SemaphoreType.DMA((2,)),
        pltpu.SemaphoreType.DMA((2,)),
    ],
)
def _sc_gather(p2, rel_emb, norm_emb, heads_lo, heads_hi, tails_lo, tails_hi,
               rels, times, out_h, out_t, out_r, out_nv, idx_v, buf,
               isem, gsem, wsem):
    c = lax.axis_index("c")
    s = lax.axis_index("s")
    wid = c * NS + s
    base = wid * B_PER_W
    idx_src = (heads_lo, heads_hi, tails_lo, tails_hi, rels, times)
    for k, idx_hbm in enumerate(idx_src):
        pltpu.async_copy(idx_hbm.at[pl.ds(base, B_PER_W)], idx_v.at[k], isem.at[k])
    plans = ((p2, 0, 1, out_h), (p2, 2, 3, out_t),
             (rel_emb, 4, None, out_r), (norm_emb, 5, None, out_nv))
    for k, (tbl, i0, i1, dst) in enumerate(plans):
        b = k % 2
        if k >= 2:
            pltpu.make_async_copy(buf.at[b], dst.at[pl.ds(base, B_PER_W)],
                                  wsem.at[b]).wait()
        pltpu.make_async_copy(idx_src[i0].at[pl.ds(base, B_PER_W)],
                              idx_v.at[i0], isem.at[i0]).wait()
        pltpu.async_copy(tbl.at[idx_v.at[i0]], buf.at[b], gsem.at[b]).wait()
        if i1 is not None:
            pltpu.make_async_copy(idx_src[i1].at[pl.ds(base, B_PER_W)],
                                  idx_v.at[i1], isem.at[i1]).wait()
            pltpu.async_copy(tbl.at[idx_v.at[i1]], buf.at[b], gsem.at[b],
                             add=True).wait()
        pltpu.async_copy(buf.at[b], dst.at[pl.ds(base, B_PER_W)], wsem.at[b])
    for b in range(2):
        pltpu.make_async_copy(buf.at[b], out_r.at[pl.ds(base, B_PER_W)],
                              wsem.at[b]).wait()


ACT_BLK = 2000


def _act_body(p_ref, w_ref, b_ref, o_ref):
    agg = p_ref[0] + p_ref[1]
    y = lax.dot_general(agg, w_ref[...], (((1,), (1,)), ((), ())),
                        preferred_element_type=jnp.float32)
    o_ref[...] = jnp.tanh(y + b_ref[...])


_tc_act = pl.pallas_call(
    _act_body,
    grid=(N_NODES // ACT_BLK,),
    in_specs=[
        pl.BlockSpec((NC, ACT_BLK, DIM), lambda i: (0, i, 0)),
        pl.BlockSpec((DIM, DIM), lambda i: (0, 0)),
        pl.BlockSpec((1, DIM), lambda i: (0, 0)),
    ],
    out_specs=pl.BlockSpec((ACT_BLK, DIM), lambda i: (i, 0)),
    out_shape=jax.ShapeDtypeStruct((N_NODES, DIM), jnp.float32),
)

SCORE_BLK = 4096


def _l2n(e, eps=1e-12):
    n = jnp.sqrt(jnp.sum(e * e, axis=-1, keepdims=True))
    return e / jnp.maximum(n, eps)


def _score_body(h_ref, t_ref, r_ref, nv_ref, w_ref, b_ref, o_ref):
    def act(p_ref):
        y = lax.dot_general(p_ref[...], w_ref[...], (((1,), (1,)), ((), ())),
                            preferred_element_type=jnp.float32)
        return jnp.tanh(y + b_ref[...])

    nvn = _l2n(nv_ref[...])

    def proj(e):
        return e - jnp.sum(nvn * e, axis=-1, keepdims=True) * nvn

    h = _l2n(proj(act(h_ref)))
    r = _l2n(proj(r_ref[...]))
    t = _l2n(proj(act(t_ref)))
    d = h + r - t
    o_ref[...] = jnp.sqrt(jnp.sum(d * d, axis=-1, keepdims=True))


_tc_score = pl.pallas_call(
    _score_body,
    grid=(BATCH // SCORE_BLK,),
    in_specs=[
        pl.BlockSpec((SCORE_BLK, DIM), lambda i: (i, 0)),
        pl.BlockSpec((SCORE_BLK, DIM), lambda i: (i, 0)),
        pl.BlockSpec((SCORE_BLK, DIM), lambda i: (i, 0)),
        pl.BlockSpec((SCORE_BLK, DIM), lambda i: (i, 0)),
        pl.BlockSpec((DIM, DIM), lambda i: (0, 0)),
        pl.BlockSpec((1, DIM), lambda i: (0, 0)),
    ],
    out_specs=pl.BlockSpec((SCORE_BLK, 1), lambda i: (i, 0)),
    out_shape=jax.ShapeDtypeStruct((BATCH, 1), jnp.float32),
)


def kernel(x, edge_index, head_batched, rel_batched, tail_batched, time_batched,
           gcn_W, gcn_b, rel_emb, norm_emb):
    src = edge_index[0]
    dst = edge_index[1]
    pad = E_PAD - N_EDGES
    pad_i = jnp.arange(pad, dtype=jnp.int32)
    src_pad = pad_i % N_NODES
    dst_pad = N_NODES + pad_i % (NPAD - N_NODES)
    pads = jnp.stack([src_pad, dst_pad]).reshape(2, E_PAD - N_EDGES)
    edges = jnp.concatenate(
        [edge_index.reshape(2, N_EDGES // K, K),
         pads.reshape(2, (E_PAD - N_EDGES) // K, K)], axis=1)
    zeros = jnp.zeros((NPAD, DIM), jnp.float32)
    b2 = gcn_b.reshape(1, DIM)

    p1 = _sc_segment_sum(x, edges, zeros)
    h1 = _tc_act(p1, gcn_W, b2)
    p2 = _sc_segment_sum(h1, edges, zeros)

    h_agg, t_agg, r, nv = _sc_gather(
        p2.reshape(NC * NPAD, DIM), rel_emb, norm_emb,
        head_batched, head_batched + NPAD, tail_batched, tail_batched + NPAD,
        rel_batched, time_batched)
    return _tc_score(h_agg, t_agg, r, nv, gcn_W, b2).reshape(-1)
